# baseline (speedup 1.0000x reference)
import jax
import jax.numpy as jnp
from jax import lax
from jax.experimental import pallas as pl
from jax.experimental.pallas import tpu as pltpu
from jax.experimental.pallas import tpu_sc as plsc

M = 2097152
N = 16384
NW = 32
RPW = N // NW
CH = 2048
KPC = CH // 16
NCHUNK = M // CH
CU_PAD = N + 8
T_THRESH = 1e-4
BG = 1.0

_I16 = lambda: lax.iota(jnp.int32, 16)


def _splat_i(x):
    return jnp.full((16,), x, jnp.int32)


def _splat_f(x):
    return jnp.full((16,), x, jnp.float32)


def _sload(ref, i):
    v = plsc.load_gather(ref, [_splat_i(i)])
    return v[0]


def _body(sig_hbm, ts_hbm, rgb_hbm, cu_hbm,
          w_hbm, ws_hbm, d_hbm, img_hbm,
          cu_ref, sig_ref, ts_ref, rgb_ref, w_ref,
          wc_ref, aw_ref, ws_ref, d_ref, img_ref, sems, wsem):
    wid = lax.axis_index("s") * 2 + lax.axis_index("c")
    r0 = wid * RPW
    r1 = r0 + RPW

    pltpu.sync_copy(cu_hbm, cu_ref)

    S = _sload(cu_ref, r0)
    E = _sload(cu_ref, r1)
    jH = jnp.minimum(S // CH, NCHUNK - 1)
    jA = (S + CH - 1) // CH
    jB = (E + CH - 1) // CH
    jB2 = jnp.maximum(jB, jH + 1)

    idx = _I16()

    def in_dma(j, slot):
        off = pl.multiple_of(j * CH, CH)
        blk = pl.multiple_of(j * (CH // 128), CH // 128)
        return (
            pltpu.make_async_copy(sig_hbm.at[pl.ds(off, CH)],
                                  sig_ref.at[slot], sems.at[slot, 0]),
            pltpu.make_async_copy(ts_hbm.at[pl.ds(blk, CH // 128)],
                                  ts_ref.at[slot], sems.at[slot, 1]),
            pltpu.make_async_copy(rgb_hbm.at[pl.ds(blk, CH // 128)],
                                  rgb_ref.at[slot], sems.at[slot, 2]),
        )

    def start_in(j, slot):
        for c in in_dma(j, slot):
            c.start()

    def wait_in(j, slot):
        for c in in_dma(j, slot):
            c.wait()

    def pass_a(o16, slot, carryE):
        b = o16 // 128
        l = pl.multiple_of(o16 - b * 128, 16)
        sig = sig_ref[slot, pl.ds(o16, 16)]
        dtv = ts_ref[slot, b, 1, pl.ds(l, 16)]
        tau = sig * dtv
        inc = plsc.cumsum(tau) + carryE
        wi = jnp.exp(-inc)
        wc = jnp.exp(-(inc - tau))
        wc_ref[pl.ds(o16, 16)] = wc
        aw_ref[pl.ds(o16, 16)] = wc - wi
        return inc[15]

    def process_b(g, o16, slot, off, decay, st):
        (r, cu_r, cu_r1, g_r, aW, aD, aR, aG, aB, w_acc) = st
        gi = idx + g
        m = (gi >= cu_r) & (gi < cu_r1)
        b = o16 // 128
        l = pl.multiple_of(o16 - b * 128, 16)
        wc = wc_ref[pl.ds(o16, 16)]
        aw = aw_ref[pl.ds(o16, 16)]
        T = wc * g_r
        w_r = jnp.where(m & (T >= T_THRESH), aw * g_r, 0.0)
        w_acc = w_acc + w_r
        tv = ts_ref[slot, b, 0, pl.ds(l, 16)]
        rv = rgb_ref[slot, b, 0, pl.ds(l, 16)]
        gv = rgb_ref[slot, b, 1, pl.ds(l, 16)]
        bv = rgb_ref[slot, b, 2, pl.ds(l, 16)]
        aW = aW + w_r
        aD = aD + w_r * tv
        aR = aR + w_r * rv
        aG = aG + w_r * gv
        aB = aB + w_r * bv
        ends = cu_r1 <= g + 16

        def slow(_):
            downer = (r >= r0) & (r < r1)
            rl = jnp.clip(r - r0, 0, RPW - 1)
            smask = (idx == 0) & downer
            plsc.store_scatter(ws_ref, [_splat_i(rl)], _splat_f(jnp.sum(aW)),
                               mask=smask)
            plsc.store_scatter(d_ref, [_splat_i(rl)], _splat_f(jnp.sum(aD)),
                               mask=smask)
            rgbv = jnp.where(idx == 0, jnp.sum(aR),
                             jnp.where(idx == 1, jnp.sum(aG), jnp.sum(aB)))
            plsc.store_scatter(img_ref, [_splat_i(rl), idx], rgbv,
                               mask=(idx < 3) & downer)
            nxt = _sload(cu_ref, jnp.minimum(r + 2, N))
            sl = cu_r1 - off
            wcs = plsc.load_gather(wc_ref, [_splat_i(jnp.clip(sl, 0,
                                                              CH - 1))])[0]
            den = jnp.where(sl >= CH, decay, wcs)
            g_n = (1.0 / _splat_f(den))[0]
            zv = _splat_f(0.0)
            return (r + 1, cu_r1, nxt, g_n,
                    zv, zv, zv, zv, zv, w_acc)

        def fast(_):
            return (r, cu_r, cu_r1, g_r,
                    aW, aD, aR, aG, aB, w_acc)

        st = lax.cond(ends, slow, fast, 0)
        return st, ends

    def chunk_body(j, st):
        slot = (j - jH) % 2
        nslot = 1 - slot

        @pl.when(j + 1 < jB2)
        def _():
            start_in(j + 1, nslot)

        wait_in(j, slot)
        off = pl.multiple_of(j * CH, CH)

        @pl.when((j - jH >= 2) & (jnp.maximum(j - 2, 0) >= jA))
        def _():
            offp = pl.multiple_of(jnp.maximum(j - 2, 0) * CH, CH)
            pltpu.make_async_copy(w_ref.at[slot],
                                  w_hbm.at[pl.ds(offp, CH)],
                                  wsem.at[slot]).wait()

        tot = lax.fori_loop(
            0, KPC, lambda k, c: pass_a(k * 16, slot, c), jnp.float32(0.0))
        decay = jnp.exp(-_splat_f(tot))[0]

        def vreg_body(k, st):
            o16 = k * 16
            g = off + o16
            st = st[:9] + (_splat_f(0.0),)
            st, ends = process_b(g, o16, slot, off, decay, st)

            def wcond(c):
                s, e = c
                return e & (s[0] < N)

            def wbody(c):
                s, _ = c
                return process_b(g, o16, slot, off, decay, s)

            st, _ = lax.while_loop(wcond, wbody, (st, ends))
            w_ref[slot, pl.ds(o16, 16)] = st[9]
            return st

        st = lax.fori_loop(0, KPC, vreg_body, st)
        st = (st[0], st[1], st[2], st[3] * decay) + st[4:]

        @pl.when((j >= jA) & (j < jB))
        def _():
            pltpu.make_async_copy(w_ref.at[slot],
                                  w_hbm.at[pl.ds(off, CH)],
                                  wsem.at[slot]).start()

        return st

    st0 = (r0 - 1, S, S, jnp.float32(1.0),
           _splat_f(0.0), _splat_f(0.0), _splat_f(0.0), _splat_f(0.0),
           _splat_f(0.0), _splat_f(0.0))
    start_in(jH, 0)
    lax.fori_loop(jH, jB2, chunk_body, st0)

    def drain(jj):
        @pl.when((jj >= jH) & (jj >= jA) & (jj < jB))
        def _():
            offp = pl.multiple_of(jnp.maximum(jj, 0) * CH, CH)
            slotp = (jj - jH) % 2
            pltpu.make_async_copy(w_ref.at[slotp],
                                  w_hbm.at[pl.ds(offp, CH)],
                                  wsem.at[slotp]).wait()

    drain(jB2 - 2)
    drain(jB2 - 1)

    def blend_body(q, _):
        qb = q * 16
        flat = idx + qb
        row = flat // 3
        col = flat - row * 3
        v = plsc.load_gather(img_ref, [row, col])
        wsv = plsc.load_gather(ws_ref, [row])
        plsc.store_scatter(img_ref, [row, col], v + (1.0 - wsv) * BG)
        return 0

    lax.fori_loop(0, RPW * 3 // 16, blend_body, 0)

    pltpu.sync_copy(ws_ref, ws_hbm.at[pl.ds(r0, RPW)])
    pltpu.sync_copy(d_ref, d_hbm.at[pl.ds(r0, RPW)])
    pltpu.sync_copy(img_ref, img_hbm.at[pl.ds(r0, RPW)])


@jax.jit
def kernel(sigmas, rgbs, ts, cu_seqlens):
    cu_pad = jnp.concatenate(
        [cu_seqlens, jnp.full((CU_PAD - N - 1,), M, jnp.int32)])
    ts_b = ts.reshape(M // 128, 128, 2).transpose(0, 2, 1)
    rgb_b = rgbs.reshape(M // 128, 128, 3).transpose(0, 2, 1)
    mesh = plsc.VectorSubcoreMesh(core_axis_name="c", subcore_axis_name="s")
    f = pl.kernel(
        _body,
        out_type=(
            jax.ShapeDtypeStruct((M,), jnp.float32),
            jax.ShapeDtypeStruct((N,), jnp.float32),
            jax.ShapeDtypeStruct((N,), jnp.float32),
            jax.ShapeDtypeStruct((N, 3), jnp.float32),
        ),
        mesh=mesh,
        compiler_params=pltpu.CompilerParams(
            needs_layout_passes=False, use_tc_tiling_on_sc=False),
        scratch_types=[
            pltpu.VMEM((CU_PAD,), jnp.int32),
            pltpu.VMEM((2, CH), jnp.float32),
            pltpu.VMEM((2, CH // 128, 2, 128), jnp.float32),
            pltpu.VMEM((2, CH // 128, 3, 128), jnp.float32),
            pltpu.VMEM((2, CH), jnp.float32),
            pltpu.VMEM((CH,), jnp.float32),
            pltpu.VMEM((CH,), jnp.float32),
            pltpu.VMEM((RPW,), jnp.float32),
            pltpu.VMEM((RPW,), jnp.float32),
            pltpu.VMEM((RPW, 3), jnp.float32),
            pltpu.SemaphoreType.DMA((2, 3)),
            pltpu.SemaphoreType.DMA((2,)),
        ],
    )
    return f(sigmas, ts_b, rgb_b, cu_pad)


# speedup vs baseline: 1.8004x; 1.8004x over previous
import jax
import jax.numpy as jnp
from jax import lax
from jax.experimental import pallas as pl
from jax.experimental.pallas import tpu as pltpu
from jax.experimental.pallas import tpu_sc as plsc

M = 2097152
N = 16384
NW = 32
RPW = N // NW
CH = 2048
KPC = CH // 16
NCHUNK = M // CH
CU_PAD = N + 8
T_THRESH = 1e-4
BG = 1.0

_I16 = lambda: lax.iota(jnp.int32, 16)


def _splat_i(x):
    return jnp.full((16,), x, jnp.int32)


def _splat_f(x):
    return jnp.full((16,), x, jnp.float32)


def _sload(ref, i):
    v = plsc.load_gather(ref, [_splat_i(i)])
    return v[0]


def _body(sig_hbm, ts_hbm, rgb_hbm, cu_hbm,
          w_hbm, ws_hbm, d_hbm, img_hbm,
          cu_ref, sig_ref, ts_ref, rgb_ref, w_ref,
          wc_ref, aw_ref, ws_ref, d_ref, img_ref, sems, wsem):
    wid = lax.axis_index("s") * 2 + lax.axis_index("c")
    r0 = wid * RPW
    r1 = r0 + RPW

    pltpu.sync_copy(cu_hbm, cu_ref)

    S = _sload(cu_ref, r0)
    E = _sload(cu_ref, r1)
    jH = jnp.minimum(S // CH, NCHUNK - 1)
    jA = (S + CH - 1) // CH
    jB = (E + CH - 1) // CH
    jB2 = jnp.maximum(jB, jH + 1)

    idx = _I16()

    def in_dma(j, slot):
        off = pl.multiple_of(j * CH, CH)
        blk = pl.multiple_of(j * (CH // 128), CH // 128)
        return (
            pltpu.make_async_copy(sig_hbm.at[pl.ds(off, CH)],
                                  sig_ref.at[slot], sems.at[slot, 0]),
            pltpu.make_async_copy(ts_hbm.at[pl.ds(blk, CH // 128)],
                                  ts_ref.at[slot], sems.at[slot, 1]),
            pltpu.make_async_copy(rgb_hbm.at[pl.ds(blk, CH // 128)],
                                  rgb_ref.at[slot], sems.at[slot, 2]),
        )

    def start_in(j, slot):
        for c in in_dma(j, slot):
            c.start()

    def wait_in(j, slot):
        for c in in_dma(j, slot):
            c.wait()

    def pass_a(o16, slot, carryE):
        b = o16 // 128
        l = pl.multiple_of(o16 - b * 128, 16)
        sig = sig_ref[slot, pl.ds(o16, 16)]
        dtv = ts_ref[slot, b, 1, pl.ds(l, 16)]
        tau = sig * dtv
        inc = plsc.cumsum(tau) + carryE
        wi = jnp.exp(-inc)
        wc = jnp.exp(-(inc - tau))
        wc_ref[pl.ds(o16, 16)] = wc
        aw_ref[pl.ds(o16, 16)] = wc - wi
        w_ref[slot, pl.ds(o16, 16)] = _splat_f(0.0)
        return inc[15]

    def ray_walk(slot, off, decay, st):
        def wcond(c):
            return c[0] < KPC

        def wbody(c):
            (k, r, cu_r, cu_r1, g_r, aW, aD, aR, aG, aB) = c
            kend = jnp.clip((cu_r1 - off + 15) // 16, k, KPC)

            def ibody(kk, acc):
                (aW, aD, aR, aG, aB) = acc
                o16 = kk * 16
                b = o16 // 128
                l = pl.multiple_of(o16 - b * 128, 16)
                gi = idx + (off + o16)
                m = (gi >= cu_r) & (gi < cu_r1)
                wc = wc_ref[pl.ds(o16, 16)]
                aw = aw_ref[pl.ds(o16, 16)]
                T = wc * g_r
                w_r = jnp.where(m & (T >= T_THRESH), aw * g_r, 0.0)
                plsc.addupdate(w_ref.at[slot, pl.ds(o16, 16)], w_r)
                tv = ts_ref[slot, b, 0, pl.ds(l, 16)]
                rv = rgb_ref[slot, b, 0, pl.ds(l, 16)]
                gv = rgb_ref[slot, b, 1, pl.ds(l, 16)]
                bv = rgb_ref[slot, b, 2, pl.ds(l, 16)]
                return (aW + w_r, aD + w_r * tv, aR + w_r * rv,
                        aG + w_r * gv, aB + w_r * bv)

            (aW, aD, aR, aG, aB) = lax.fori_loop(
                k, kend, ibody, (aW, aD, aR, aG, aB))
            ends = cu_r1 <= off + CH

            def slow(_):
                downer = (r >= r0) & (r < r1)
                rl = jnp.clip(r - r0, 0, RPW - 1)
                smask = (idx == 0) & downer
                plsc.store_scatter(ws_ref, [_splat_i(rl)],
                                   _splat_f(jnp.sum(aW)), mask=smask)
                plsc.store_scatter(d_ref, [_splat_i(rl)],
                                   _splat_f(jnp.sum(aD)), mask=smask)
                rgbv = jnp.where(idx == 0, jnp.sum(aR),
                                 jnp.where(idx == 1, jnp.sum(aG),
                                           jnp.sum(aB)))
                plsc.store_scatter(img_ref, [_splat_i(rl), idx], rgbv,
                                   mask=(idx < 3) & downer)
                nxt = _sload(cu_ref, jnp.minimum(r + 2, N))
                sl = cu_r1 - off
                wcs = plsc.load_gather(
                    wc_ref, [_splat_i(jnp.clip(sl, 0, CH - 1))])[0]
                den = jnp.where(sl >= CH, decay, wcs)
                g_n = (1.0 / _splat_f(den))[0]
                zv = _splat_f(0.0)
                kn = jnp.minimum(jnp.maximum(sl, 0) // 16, KPC)
                return (kn, r + 1, cu_r1, nxt, g_n, zv, zv, zv, zv, zv)

            def cont(_):
                return (KPC, r, cu_r, cu_r1, g_r, aW, aD, aR, aG, aB)

            return lax.cond(ends, slow, cont, 0)

        (k, r, cu_r, cu_r1, g_r, aW, aD, aR, aG, aB) = lax.while_loop(
            wcond, wbody, (0,) + st)
        return (r, cu_r, cu_r1, g_r * decay, aW, aD, aR, aG, aB)

    def chunk_body(j, st):
        slot = (j - jH) % 2
        nslot = 1 - slot

        @pl.when(j + 1 < jB2)
        def _():
            start_in(j + 1, nslot)

        wait_in(j, slot)
        off = pl.multiple_of(j * CH, CH)

        @pl.when((j - jH >= 2) & (jnp.maximum(j - 2, 0) >= jA))
        def _():
            offp = pl.multiple_of(jnp.maximum(j - 2, 0) * CH, CH)
            pltpu.make_async_copy(w_ref.at[slot],
                                  w_hbm.at[pl.ds(offp, CH)],
                                  wsem.at[slot]).wait()

        tot = lax.fori_loop(
            0, KPC, lambda k, c: pass_a(k * 16, slot, c), jnp.float32(0.0))
        decay = jnp.exp(-_splat_f(tot))[0]
        st = ray_walk(slot, off, decay, st)

        @pl.when((j >= jA) & (j < jB))
        def _():
            pltpu.make_async_copy(w_ref.at[slot],
                                  w_hbm.at[pl.ds(off, CH)],
                                  wsem.at[slot]).start()

        return st

    st0 = (r0 - 1, S, S, jnp.float32(1.0),
           _splat_f(0.0), _splat_f(0.0), _splat_f(0.0), _splat_f(0.0),
           _splat_f(0.0))
    start_in(jH, 0)
    lax.fori_loop(jH, jB2, chunk_body, st0)

    def drain(jj):
        @pl.when((jj >= jH) & (jj >= jA) & (jj < jB))
        def _():
            offp = pl.multiple_of(jnp.maximum(jj, 0) * CH, CH)
            slotp = (jj - jH) % 2
            pltpu.make_async_copy(w_ref.at[slotp],
                                  w_hbm.at[pl.ds(offp, CH)],
                                  wsem.at[slotp]).wait()

    drain(jB2 - 2)
    drain(jB2 - 1)

    def blend_body(q, _):
        qb = q * 16
        flat = idx + qb
        row = flat // 3
        col = flat - row * 3
        v = plsc.load_gather(img_ref, [row, col])
        wsv = plsc.load_gather(ws_ref, [row])
        plsc.store_scatter(img_ref, [row, col], v + (1.0 - wsv) * BG)
        return 0

    lax.fori_loop(0, RPW * 3 // 16, blend_body, 0)

    pltpu.sync_copy(ws_ref, ws_hbm.at[pl.ds(r0, RPW)])
    pltpu.sync_copy(d_ref, d_hbm.at[pl.ds(r0, RPW)])
    pltpu.sync_copy(img_ref, img_hbm.at[pl.ds(r0, RPW)])


@jax.jit
def kernel(sigmas, rgbs, ts, cu_seqlens):
    cu_pad = jnp.concatenate(
        [cu_seqlens, jnp.full((CU_PAD - N - 1,), M, jnp.int32)])
    ts_b = ts.reshape(M // 128, 128, 2).transpose(0, 2, 1)
    rgb_b = rgbs.reshape(M // 128, 128, 3).transpose(0, 2, 1)
    mesh = plsc.VectorSubcoreMesh(core_axis_name="c", subcore_axis_name="s")
    f = pl.kernel(
        _body,
        out_type=(
            jax.ShapeDtypeStruct((M,), jnp.float32),
            jax.ShapeDtypeStruct((N,), jnp.float32),
            jax.ShapeDtypeStruct((N,), jnp.float32),
            jax.ShapeDtypeStruct((N, 3), jnp.float32),
        ),
        mesh=mesh,
        compiler_params=pltpu.CompilerParams(
            needs_layout_passes=False, use_tc_tiling_on_sc=False),
        scratch_types=[
            pltpu.VMEM((CU_PAD,), jnp.int32),
            pltpu.VMEM((2, CH), jnp.float32),
            pltpu.VMEM((2, CH // 128, 2, 128), jnp.float32),
            pltpu.VMEM((2, CH // 128, 3, 128), jnp.float32),
            pltpu.VMEM((2, CH), jnp.float32),
            pltpu.VMEM((CH,), jnp.float32),
            pltpu.VMEM((CH,), jnp.float32),
            pltpu.VMEM((RPW,), jnp.float32),
            pltpu.VMEM((RPW,), jnp.float32),
            pltpu.VMEM((RPW, 3), jnp.float32),
            pltpu.SemaphoreType.DMA((2, 3)),
            pltpu.SemaphoreType.DMA((2,)),
        ],
    )
    return f(sigmas, ts_b, rgb_b, cu_pad)


# speedup vs baseline: 2.6763x; 1.4865x over previous
import jax
import jax.numpy as jnp
from jax import lax
from jax.experimental import pallas as pl
from jax.experimental.pallas import tpu as pltpu
from jax.experimental.pallas import tpu_sc as plsc

M = 2097152
N = 16384
NW = 32
RPW = N // NW
CH = 2048
KPC = CH // 16
NCHUNK = M // CH
CU_PAD = N + 8
T_THRESH = 1e-4
BG = 1.0

_I16 = lambda: lax.iota(jnp.int32, 16)


def _splat_i(x):
    return jnp.full((16,), x, jnp.int32)


def _splat_f(x):
    return jnp.full((16,), x, jnp.float32)


def _sload(ref, i):
    v = plsc.load_gather(ref, [_splat_i(i)])
    return v[0]


def _body(sig_hbm, ts_hbm, rgb_hbm, cu_hbm,
          w_hbm, ws_hbm, d_hbm, img_hbm,
          cu_ref, sig_ref, ts_ref, rgb_ref, w_ref,
          wc_ref, aw_ref, cb_ref, ws_ref, d_ref, img_ref, sems, wsem):
    wid = lax.axis_index("s") * 2 + lax.axis_index("c")
    r0 = wid * RPW
    r1 = r0 + RPW

    pltpu.sync_copy(cu_hbm, cu_ref)

    S = _sload(cu_ref, r0)
    E = _sload(cu_ref, r1)
    jH = jnp.minimum(S // CH, NCHUNK - 1)
    jA = (S + CH - 1) // CH
    jB = (E + CH - 1) // CH
    jB2 = jnp.maximum(jB, jH + 1)

    idx = _I16()

    def in_dma(j, slot):
        off = pl.multiple_of(j * CH, CH)
        blk = pl.multiple_of(j * (CH // 128), CH // 128)
        return (
            pltpu.make_async_copy(sig_hbm.at[pl.ds(off, CH)],
                                  sig_ref.at[slot], sems.at[slot, 0]),
            pltpu.make_async_copy(ts_hbm.at[pl.ds(blk, CH // 128)],
                                  ts_ref.at[slot], sems.at[slot, 1]),
            pltpu.make_async_copy(rgb_hbm.at[pl.ds(blk, CH // 128)],
                                  rgb_ref.at[slot], sems.at[slot, 2]),
        )

    def start_in(j, slot):
        for c in in_dma(j, slot):
            c.start()

    def wait_in(j, slot):
        for c in in_dma(j, slot):
            c.wait()

    def pass_a(slot):
        @plsc.parallel_loop(0, KPC, unroll=4)
        def _(k):
            o16 = pl.multiple_of(k * 16, 16)
            b = o16 // 128
            l = pl.multiple_of(o16 - b * 128, 16)
            sig = sig_ref[slot, pl.ds(o16, 16)]
            dtv = ts_ref[slot, b, 1, pl.ds(l, 16)]
            tau = sig * dtv
            wc_ref[pl.ds(o16, 16)] = plsc.cumsum(tau)
            aw_ref[pl.ds(o16, 16)] = tau
            w_ref[slot, pl.ds(o16, 16)] = _splat_f(0.0)

        carry_blk = jnp.float32(0.0)
        for i in range(KPC // 16):
            p = _I16() * 16 + (256 * i + 15)
            tvec = plsc.load_gather(wc_ref, [p])
            cum = plsc.cumsum(tvec)
            cb_ref[pl.ds(16 * i, 16)] = cum - tvec + carry_blk
            carry_blk = carry_blk + cum[15]

        @plsc.parallel_loop(0, KPC, unroll=4)
        def _(k):
            o16 = pl.multiple_of(k * 16, 16)
            tau = aw_ref[pl.ds(o16, 16)]
            inc = wc_ref[pl.ds(o16, 16)] + plsc.load_gather(
                cb_ref, [_splat_i(k)])
            wi = jnp.exp(-inc)
            wc = jnp.exp(-(inc - tau))
            wc_ref[pl.ds(o16, 16)] = wc
            aw_ref[pl.ds(o16, 16)] = wc - wi

        return carry_blk

    def ray_walk(slot, off, decay, st):
        def wcond(c):
            return c[0] < KPC

        def wbody(c):
            (k, r, cu_r, cu_r1, g_r, aW, aD, aR, aG, aB) = c
            kend = jnp.clip((cu_r1 - off + 15) // 16, k, KPC)

            def ibody(kk, acc):
                (aW, aD, aR, aG, aB) = acc
                o16 = kk * 16
                b = o16 // 128
                l = pl.multiple_of(o16 - b * 128, 16)
                gi = idx + (off + o16)
                m = (gi >= cu_r) & (gi < cu_r1)
                wc = wc_ref[pl.ds(o16, 16)]
                aw = aw_ref[pl.ds(o16, 16)]
                T = wc * g_r
                w_r = jnp.where(m & (T >= T_THRESH), aw * g_r, 0.0)
                plsc.addupdate(w_ref.at[slot, pl.ds(o16, 16)], w_r)
                tv = ts_ref[slot, b, 0, pl.ds(l, 16)]
                rv = rgb_ref[slot, b, 0, pl.ds(l, 16)]
                gv = rgb_ref[slot, b, 1, pl.ds(l, 16)]
                bv = rgb_ref[slot, b, 2, pl.ds(l, 16)]
                return (aW + w_r, aD + w_r * tv, aR + w_r * rv,
                        aG + w_r * gv, aB + w_r * bv)

            (aW, aD, aR, aG, aB) = lax.fori_loop(
                k, kend, ibody, (aW, aD, aR, aG, aB))
            ends = cu_r1 <= off + CH

            def slow(_):
                downer = (r >= r0) & (r < r1)
                rl = jnp.clip(r - r0, 0, RPW - 1)
                smask = (idx == 0) & downer
                plsc.store_scatter(ws_ref, [_splat_i(rl)],
                                   _splat_f(jnp.sum(aW)), mask=smask)
                plsc.store_scatter(d_ref, [_splat_i(rl)],
                                   _splat_f(jnp.sum(aD)), mask=smask)
                rgbv = jnp.where(idx == 0, jnp.sum(aR),
                                 jnp.where(idx == 1, jnp.sum(aG),
                                           jnp.sum(aB)))
                plsc.store_scatter(img_ref, [_splat_i(rl), idx], rgbv,
                                   mask=(idx < 3) & downer)
                nxt = _sload(cu_ref, jnp.minimum(r + 2, N))
                sl = cu_r1 - off
                wcs = plsc.load_gather(
                    wc_ref, [_splat_i(jnp.clip(sl, 0, CH - 1))])[0]
                den = jnp.where(sl >= CH, decay, wcs)
                g_n = (1.0 / _splat_f(den))[0]
                zv = _splat_f(0.0)
                kn = jnp.minimum(jnp.maximum(sl, 0) // 16, KPC)
                return (kn, r + 1, cu_r1, nxt, g_n, zv, zv, zv, zv, zv)

            def cont(_):
                return (KPC, r, cu_r, cu_r1, g_r, aW, aD, aR, aG, aB)

            return lax.cond(ends, slow, cont, 0)

        (k, r, cu_r, cu_r1, g_r, aW, aD, aR, aG, aB) = lax.while_loop(
            wcond, wbody, (0,) + st)
        return (r, cu_r, cu_r1, g_r * decay, aW, aD, aR, aG, aB)

    def chunk_body(j, st):
        slot = (j - jH) % 2
        nslot = 1 - slot

        @pl.when(j + 1 < jB2)
        def _():
            start_in(j + 1, nslot)

        wait_in(j, slot)
        off = pl.multiple_of(j * CH, CH)

        @pl.when((j - jH >= 2) & (jnp.maximum(j - 2, 0) >= jA))
        def _():
            offp = pl.multiple_of(jnp.maximum(j - 2, 0) * CH, CH)
            pltpu.make_async_copy(w_ref.at[slot],
                                  w_hbm.at[pl.ds(offp, CH)],
                                  wsem.at[slot]).wait()

        tot = pass_a(slot)
        decay = jnp.exp(-_splat_f(tot))[0]
        st = ray_walk(slot, off, decay, st)

        @pl.when((j >= jA) & (j < jB))
        def _():
            pltpu.make_async_copy(w_ref.at[slot],
                                  w_hbm.at[pl.ds(off, CH)],
                                  wsem.at[slot]).start()

        return st

    st0 = (r0 - 1, S, S, jnp.float32(1.0),
           _splat_f(0.0), _splat_f(0.0), _splat_f(0.0), _splat_f(0.0),
           _splat_f(0.0))
    start_in(jH, 0)
    lax.fori_loop(jH, jB2, chunk_body, st0)

    def drain(jj):
        @pl.when((jj >= jH) & (jj >= jA) & (jj < jB))
        def _():
            offp = pl.multiple_of(jnp.maximum(jj, 0) * CH, CH)
            slotp = (jj - jH) % 2
            pltpu.make_async_copy(w_ref.at[slotp],
                                  w_hbm.at[pl.ds(offp, CH)],
                                  wsem.at[slotp]).wait()

    drain(jB2 - 2)
    drain(jB2 - 1)

    def blend_body(q, _):
        qb = q * 16
        flat = idx + qb
        row = flat // 3
        col = flat - row * 3
        v = plsc.load_gather(img_ref, [row, col])
        wsv = plsc.load_gather(ws_ref, [row])
        plsc.store_scatter(img_ref, [row, col], v + (1.0 - wsv) * BG)
        return 0

    lax.fori_loop(0, RPW * 3 // 16, blend_body, 0)

    pltpu.sync_copy(ws_ref, ws_hbm.at[pl.ds(r0, RPW)])
    pltpu.sync_copy(d_ref, d_hbm.at[pl.ds(r0, RPW)])
    pltpu.sync_copy(img_ref, img_hbm.at[pl.ds(r0, RPW)])


@jax.jit
def kernel(sigmas, rgbs, ts, cu_seqlens):
    cu_pad = jnp.concatenate(
        [cu_seqlens, jnp.full((CU_PAD - N - 1,), M, jnp.int32)])
    ts_b = ts.reshape(M // 128, 128, 2).transpose(0, 2, 1)
    rgb_b = rgbs.reshape(M // 128, 128, 3).transpose(0, 2, 1)
    mesh = plsc.VectorSubcoreMesh(core_axis_name="c", subcore_axis_name="s")
    f = pl.kernel(
        _body,
        out_type=(
            jax.ShapeDtypeStruct((M,), jnp.float32),
            jax.ShapeDtypeStruct((N,), jnp.float32),
            jax.ShapeDtypeStruct((N,), jnp.float32),
            jax.ShapeDtypeStruct((N, 3), jnp.float32),
        ),
        mesh=mesh,
        compiler_params=pltpu.CompilerParams(
            needs_layout_passes=False, use_tc_tiling_on_sc=False),
        scratch_types=[
            pltpu.VMEM((CU_PAD,), jnp.int32),
            pltpu.VMEM((2, CH), jnp.float32),
            pltpu.VMEM((2, CH // 128, 2, 128), jnp.float32),
            pltpu.VMEM((2, CH // 128, 3, 128), jnp.float32),
            pltpu.VMEM((2, CH), jnp.float32),
            pltpu.VMEM((CH,), jnp.float32),
            pltpu.VMEM((CH,), jnp.float32),
            pltpu.VMEM((KPC,), jnp.float32),
            pltpu.VMEM((RPW,), jnp.float32),
            pltpu.VMEM((RPW,), jnp.float32),
            pltpu.VMEM((RPW, 3), jnp.float32),
            pltpu.SemaphoreType.DMA((2, 3)),
            pltpu.SemaphoreType.DMA((2,)),
        ],
    )
    return f(sigmas, ts_b, rgb_b, cu_pad)
